# baseline (speedup 1.0000x reference)
import functools

import jax
import jax.numpy as jnp
from jax import lax
from jax.experimental import pallas as pl
from jax.experimental.pallas import tpu as pltpu
from jax.experimental.pallas import tpu_sc as plsc

N_ROWS = 262144
N_COLS = 128
NUM_CORES = 2
NUM_SUBCORES = 16
NUM_WORKERS = NUM_CORES * NUM_SUBCORES
RPW = N_ROWS // NUM_WORKERS
L = 16
NCHUNK = 16
CH = RPW // NCHUNK


def _dg(v, idx):
    return lax.gather(
        v,
        idx.reshape(L, 1),
        lax.GatherDimensionNumbers(
            offset_dims=(), collapsed_slice_dims=(0,), start_index_map=(0,)
        ),
        slice_sizes=(1,),
        mode=lax.GatherScatterMode.PROMISE_IN_BOUNDS,
    )


def _sc_body(view_hbm, out_hbm, *refs):
    (irow0, irow1, brow0, brow1, acc_v, sem0, sem1) = refs
    idx_sets = (irow0, irow1)
    buf_sets = (brow0, brow1)
    sems = (sem0, sem1)

    cid = lax.axis_index("c")
    sid = lax.axis_index("s")
    wid = sid * NUM_CORES + cid
    row0 = wid * RPW
    iot = lax.iota(jnp.int32, L)
    one = jnp.ones((L,), jnp.int32)
    swap = iot ^ 1
    lane0 = jnp.zeros((L,), jnp.int32)
    qmask = tuple((iot >> 2) == q for q in range(4))
    rots = tuple((iot - 4 * r) & 15 for r in (1, 2, 3))
    pat0 = (iot & 3) * 4
    pat2 = pat0 + 2

    def fill(c, irow):
        def body(j, carry):
            irow[pl.ds(j * L, L)] = 2 * (row0 + c * CH + j * L + iot)
            return carry

        lax.fori_loop(0, CH // L, body, 0)

    def fire(irow, brow, sem):
        pltpu.async_copy(view_hbm.at[irow], brow, sem)

    def drain(irow, brow, sem):
        pltpu.make_async_copy(view_hbm.at[irow], brow, sem).wait()

    def compute(brow, acc):
        def group(g, acc2):
            dx = lane0
            dy = lane0
            ck = lane0
            for q in range(4):
                b = g * L + 4 * q
                m = brow[b, pl.ds(32, L)]
                mc = brow[b, pl.ds(0, L)]
                for r in (1, 2, 3):
                    m = jnp.where(
                        qmask[r], _dg(brow[b + r, pl.ds(32, L)], rots[r - 1]), m
                    )
                    mc = jnp.where(
                        qmask[r], _dg(brow[b + r, pl.ds(0, L)], rots[r - 1]), mc
                    )
                d = _dg(m, swap) - m
                dx = jnp.where(qmask[q], _dg(d, pat0), dx)
                dy = jnp.where(qmask[q], _dg(d, pat2), dy)
                ck = jnp.where(qmask[q], _dg(mc, pat0), ck)
            dist_x = jnp.abs(dx)
            dist_y = jnp.abs(dy)
            go_down = dy > 1
            go_right = dx > 0
            punch = (ck % 12) < 4
            d2 = dist_y <= 2
            act = jnp.where(go_down, 5, 2)
            act = jnp.where(d2 & (dist_x > 26), jnp.where(go_right, 3, 4), act)
            act = jnp.where(d2 & (dist_x < 23), jnp.where(go_right, 4, 3), act)
            act = jnp.where(
                d2 & (dist_x >= 23) & (dist_x <= 26), jnp.where(punch, 1, 0), act
            )
            return acc2 | (one << act)

        return lax.fori_loop(0, CH // L, group, acc)

    fill(0, idx_sets[0])
    fire(idx_sets[0], buf_sets[0], sems[0])
    acc = jnp.zeros((L,), jnp.int32)
    for c in range(NCHUNK):
        if c + 1 < NCHUNK:
            nxt = (c + 1) % 2
            fill(c + 1, idx_sets[nxt])
            fire(idx_sets[nxt], buf_sets[nxt], sems[nxt])
        cur = c % 2
        drain(idx_sets[cur], buf_sets[cur], sems[cur])
        acc = compute(buf_sets[cur], acc)

    acc_v[...] = acc
    pltpu.sync_copy(acc_v, out_hbm.at[wid])


@jax.jit
def _run(ram):
    view = ram.reshape(N_ROWS * 2, 64)
    mesh = plsc.VectorSubcoreMesh(core_axis_name="c", subcore_axis_name="s")
    scratch = (
        [pltpu.VMEM((CH,), jnp.int32) for _ in range(2)]
        + [pltpu.VMEM((CH, 64), jnp.int32) for _ in range(2)]
        + [pltpu.VMEM((L,), jnp.int32)]
        + [pltpu.SemaphoreType.DMA, pltpu.SemaphoreType.DMA]
    )
    k = functools.partial(
        pl.kernel,
        mesh=mesh,
        out_type=jax.ShapeDtypeStruct((NUM_WORKERS, L), jnp.int32),
        scratch_types=scratch,
        compiler_params=pltpu.CompilerParams(use_tc_tiling_on_sc=False),
    )(_sc_body)
    masks = k(view)
    bits = (masks[:, :, None] >> jnp.arange(6, dtype=jnp.int32)) & 1
    seen = jnp.max(bits, axis=(0, 1)).astype(jnp.float32)
    logits = jnp.zeros((1, 18), dtype=jnp.float32)
    return lax.dynamic_update_slice(logits, seen.reshape(1, 6), (0, 0))


def kernel(ram):
    return _run(ram)
